# Initial kernel scaffold; baseline (speedup 1.0000x reference)
import jax
import jax.numpy as jnp
from jax.experimental import pallas as pl


def kernel(x, edge_index, batch, W0, b0, W1, b1, W2, b2, W3, b3, W4, b4):
    raise NotImplementedError("write your pallas kernel here")


# speedup vs baseline: 6.8827x; 6.8827x over previous
import functools

import jax
import jax.numpy as jnp
from jax import lax
from jax.experimental import pallas as pl
from jax.experimental.pallas import tpu as pltpu
from jax.experimental.pallas import tpu_sc as plsc

N = 10000
E = 320000
G = 64
HID = 256
HALF = 128
NC = 2
NS = 16
NP = 10240
ROWS_PER_TILE = NP // NS
DEG_PER_TILE = NP // NS
K = 80
MB = 1024
MGRID = NP // MB

_sc_mesh = plsc.VectorSubcoreMesh(core_axis_name="c", subcore_axis_name="s")


@functools.partial(
    pl.kernel,
    out_type=jax.ShapeDtypeStruct((NC, NP), jnp.float32),
    mesh=_sc_mesh,
    scratch_types=[
        pltpu.VMEM((K,), jnp.int32),
        pltpu.VMEM((K,), jnp.float32),
        pltpu.VMEM_SHARED((NP,), jnp.float32),
        pltpu.SemaphoreType.DMA,
    ],
)
def _deg_sc(col_hbm, init_hbm, deg_hbm, idx_v, ones_v, deg_sh, sem):
    del sem
    c = lax.axis_index("c")
    t = lax.axis_index("s")
    r0 = t * DEG_PER_TILE
    pltpu.sync_copy(init_hbm.at[c, pl.ds(r0, DEG_PER_TILE)],
                    deg_sh.at[pl.ds(r0, DEG_PER_TILE)])
    pltpu.sync_copy(init_hbm.at[0, pl.ds(0, K)], ones_v)
    plsc.subcore_barrier()

    ept = E // (NC * NS)
    base = (c * NS + t) * ept

    def win(w, carry):
        pltpu.sync_copy(col_hbm.at[pl.ds(base + w * K, K)], idx_v)
        pltpu.sync_copy(ones_v, deg_sh.at[idx_v], add=True)
        return carry

    lax.fori_loop(0, ept // K, win, 0)
    plsc.subcore_barrier()
    pltpu.sync_copy(deg_sh.at[pl.ds(r0, DEG_PER_TILE)],
                    deg_hbm.at[c, pl.ds(r0, DEG_PER_TILE)])


@functools.partial(
    pl.kernel,
    out_type=jax.ShapeDtypeStruct((NC, NP, HALF), jnp.float32),
    mesh=_sc_mesh,
    scratch_types=[
        pltpu.VMEM((K,), jnp.int32),
        pltpu.VMEM((K,), jnp.int32),
        pltpu.VMEM((K, HALF), jnp.float32),
        pltpu.VMEM_SHARED((NP, HALF), jnp.float32),
        pltpu.SemaphoreType.DMA,
    ],
)
def _agg_sc(y0_hbm, y1_hbm, row_hbm, col_hbm, out_hbm,
            idx_r, idx_c, rows_v, acc_sh, sem):
    c = lax.axis_index("c")
    t = lax.axis_index("s")
    r0 = t * ROWS_PER_TILE
    ept = E // NS

    def run(y_ref):
        pltpu.sync_copy(y_ref.at[pl.ds(r0, ROWS_PER_TILE)],
                        acc_sh.at[pl.ds(r0, ROWS_PER_TILE)])
        plsc.subcore_barrier()
        base = t * ept

        def win(w, carry):
            b = base + w * K
            pltpu.sync_copy(row_hbm.at[pl.ds(b, K)], idx_r)
            pltpu.async_copy(y_ref.at[idx_r], rows_v, sem).wait()
            pltpu.sync_copy(col_hbm.at[pl.ds(b, K)], idx_c)
            pltpu.sync_copy(rows_v, acc_sh.at[idx_c], add=True)
            return carry

        lax.fori_loop(0, ept // K, win, 0)
        plsc.subcore_barrier()
        pltpu.sync_copy(acc_sh.at[pl.ds(r0, ROWS_PER_TILE)],
                        out_hbm.at[c, pl.ds(r0, ROWS_PER_TILE)])

    @pl.when(c == 0)
    def _():
        run(y0_hbm)

    @pl.when(c == 1)
    def _():
        run(y1_hbm)


def _mm_first_body(x_ref, w_ref, deg_ref, y0_ref, y1_ref):
    d = deg_ref[0] + deg_ref[1]
    dinv = lax.rsqrt(d)
    z = jnp.dot(x_ref[...], w_ref[...], preferred_element_type=jnp.float32)
    y = z * dinv
    y0_ref[...] = y[:, :HALF]
    y1_ref[...] = y[:, HALF:]


def _mm_mid_body(a0_ref, a1_ref, b_ref, wa_ref, wb_ref, deg_ref,
                 y0_ref, y1_ref):
    d = deg_ref[0] + deg_ref[1]
    dinv = lax.rsqrt(d)
    b = b_ref[...]
    h0 = jnp.maximum(a0_ref[...] * dinv + b[:, :HALF], 0.0)
    h1 = jnp.maximum(a1_ref[...] * dinv + b[:, HALF:], 0.0)
    z = (jnp.dot(h0, wa_ref[...], preferred_element_type=jnp.float32)
         + jnp.dot(h1, wb_ref[...], preferred_element_type=jnp.float32))
    y = z * dinv
    y0_ref[...] = y[:, :HALF]
    y1_ref[...] = y[:, HALF:]


def _pool_body(a0_ref, a1_ref, b_ref, deg_ref, batch_ref, out_ref,
               sums, cnts):
    i = pl.program_id(0)

    @pl.when(i == 0)
    def _():
        sums[...] = jnp.zeros_like(sums)
        cnts[...] = jnp.zeros_like(cnts)

    d = deg_ref[0] + deg_ref[1]
    dinv = lax.rsqrt(d)
    b = b_ref[...]
    h0 = jnp.maximum(a0_ref[...] * dinv + b[:, :HALF], 0.0)
    h1 = jnp.maximum(a1_ref[...] * dinv + b[:, HALF:], 0.0)
    h = jnp.concatenate([h0, h1], axis=1)
    gids = lax.broadcasted_iota(jnp.int32, (1, G), 1)
    mask = (batch_ref[...] == gids).astype(jnp.float32)
    dn = (((0,), (0,)), ((), ()))
    sums[...] += lax.dot_general(mask, h, dn,
                                 preferred_element_type=jnp.float32)
    cnts[...] += lax.dot_general(mask, jnp.ones((MB, HALF), jnp.float32), dn,
                                 preferred_element_type=jnp.float32)

    @pl.when(i == MGRID - 1)
    def _():
        out_ref[...] = sums[...] / jnp.maximum(cnts[:, :1], 1.0)


def _mm_first(x, w, deg):
    return pl.pallas_call(
        _mm_first_body,
        grid=(MGRID,),
        in_specs=[
            pl.BlockSpec((MB, HALF), lambda i: (i, 0)),
            pl.BlockSpec((HALF, HID), lambda i: (0, 0)),
            pl.BlockSpec((NC, MB, 1), lambda i: (0, i, 0)),
        ],
        out_specs=[
            pl.BlockSpec((MB, HALF), lambda i: (i, 0)),
            pl.BlockSpec((MB, HALF), lambda i: (i, 0)),
        ],
        out_shape=[
            jax.ShapeDtypeStruct((NP, HALF), jnp.float32),
            jax.ShapeDtypeStruct((NP, HALF), jnp.float32),
        ],
    )(x, w, deg)


def _mm_mid(a0, a1, b, wa, wb, deg):
    return pl.pallas_call(
        _mm_mid_body,
        grid=(MGRID,),
        in_specs=[
            pl.BlockSpec((MB, HALF), lambda i: (i, 0)),
            pl.BlockSpec((MB, HALF), lambda i: (i, 0)),
            pl.BlockSpec((1, HID), lambda i: (0, 0)),
            pl.BlockSpec((HALF, HID), lambda i: (0, 0)),
            pl.BlockSpec((HALF, HID), lambda i: (0, 0)),
            pl.BlockSpec((NC, MB, 1), lambda i: (0, i, 0)),
        ],
        out_specs=[
            pl.BlockSpec((MB, HALF), lambda i: (i, 0)),
            pl.BlockSpec((MB, HALF), lambda i: (i, 0)),
        ],
        out_shape=[
            jax.ShapeDtypeStruct((NP, HALF), jnp.float32),
            jax.ShapeDtypeStruct((NP, HALF), jnp.float32),
        ],
    )(a0, a1, b, wa, wb, deg)


def _pool(a0, a1, b, deg, batch2d):
    return pl.pallas_call(
        _pool_body,
        grid=(MGRID,),
        in_specs=[
            pl.BlockSpec((MB, HALF), lambda i: (i, 0)),
            pl.BlockSpec((MB, HALF), lambda i: (i, 0)),
            pl.BlockSpec((1, HID), lambda i: (0, 0)),
            pl.BlockSpec((NC, MB, 1), lambda i: (0, i, 0)),
            pl.BlockSpec((MB, 1), lambda i: (i, 0)),
        ],
        out_specs=pl.BlockSpec((G, HID), lambda i: (0, 0)),
        out_shape=jax.ShapeDtypeStruct((G, HID), jnp.float32),
        scratch_shapes=[
            pltpu.VMEM((G, HID), jnp.float32),
            pltpu.VMEM((G, HALF), jnp.float32),
        ],
    )(a0, a1, b, deg, batch2d)


def kernel(x, edge_index, batch, W0, b0, W1, b1, W2, b2, W3, b3, W4, b4):
    row = edge_index[0]
    col = edge_index[1]
    init2 = jnp.stack([jnp.ones((NP,), jnp.float32),
                       jnp.zeros((NP,), jnp.float32)])

    deg2 = _deg_sc(col, init2)
    deg = deg2.reshape(NC, NP, 1)

    xp = jnp.pad(x, ((0, NP - N), (0, 0)))
    batch2d = jnp.pad(batch, (0, NP - N), constant_values=G).reshape(NP, 1)
    Ws = [W1, W2, W3, W4]
    bs = [b0.reshape(1, HID), b1.reshape(1, HID), b2.reshape(1, HID),
          b3.reshape(1, HID), b4.reshape(1, HID)]

    y0, y1 = _mm_first(xp, W0, deg)
    a = _agg_sc(y0, y1, row, col)
    for i in range(4):
        y0, y1 = _mm_mid(a[0], a[1], bs[i], Ws[i][:HALF], Ws[i][HALF:], deg)
        a = _agg_sc(y0, y1, row, col)
    return _pool(a[0], a[1], bs[4], deg, batch2d)


# speedup vs baseline: 18.5929x; 2.7014x over previous
import functools

import jax
import jax.numpy as jnp
from jax import lax
from jax.experimental import pallas as pl
from jax.experimental.pallas import tpu as pltpu
from jax.experimental.pallas import tpu_sc as plsc

N = 10000
E = 320000
G = 64
HID = 256
HALF = 128
NC = 2
NS = 16
NP = 10240
ROWS_PER_TILE = NP // NS
DEG_PER_TILE = NP // NS
EP = 327680
K = 128
MB = 1024
MGRID = NP // MB

_sc_mesh = plsc.VectorSubcoreMesh(core_axis_name="c", subcore_axis_name="s")


@functools.partial(
    pl.kernel,
    out_type=jax.ShapeDtypeStruct((NC, NP), jnp.float32),
    mesh=_sc_mesh,
    scratch_types=[
        pltpu.VMEM((K,), jnp.int32),
        pltpu.VMEM((K,), jnp.float32),
        pltpu.VMEM_SHARED((NP,), jnp.float32),
        pltpu.SemaphoreType.DMA,
    ],
)
def _deg_sc(col_hbm, init_hbm, deg_hbm, idx_v, ones_v, deg_sh, sem):
    del sem
    c = lax.axis_index("c")
    t = lax.axis_index("s")
    r0 = t * DEG_PER_TILE
    pltpu.sync_copy(init_hbm.at[c, pl.ds(r0, DEG_PER_TILE)],
                    deg_sh.at[pl.ds(r0, DEG_PER_TILE)])
    pltpu.sync_copy(init_hbm.at[0, pl.ds(0, K)], ones_v)
    plsc.subcore_barrier()

    wpt = NW // NC
    base = (c * NS + t) * wpt

    def win(w, carry):
        pltpu.sync_copy(col_hbm.at[base + w, 1], idx_v)
        pltpu.sync_copy(ones_v, deg_sh.at[idx_v], add=True)
        return carry

    lax.fori_loop(0, wpt, win, 0)
    plsc.subcore_barrier()
    pltpu.sync_copy(deg_sh.at[pl.ds(r0, DEG_PER_TILE)],
                    deg_hbm.at[c, pl.ds(r0, DEG_PER_TILE)])


NW = EP // (NS * K)
NI = 4
ND = 2

@functools.partial(
    pl.kernel,
    out_type=jax.ShapeDtypeStruct((NC, NP, HALF), jnp.float32),
    mesh=_sc_mesh,
    scratch_types=(
        [pltpu.VMEM((2, K), jnp.int32)] * NI
        + [pltpu.VMEM((K, HALF), jnp.float32)] * ND
        + [pltpu.SemaphoreType.DMA] * (NI + ND)
        + [pltpu.VMEM_SHARED((NP, HALF), jnp.float32)]
    ),
)
def _agg_sc(y0_hbm, y1_hbm, rc_hbm, out_hbm, *rest):
    idx = rest[:NI]
    bufs = rest[NI:NI + ND]
    isem = rest[NI + ND:2 * NI + ND]
    gsem = rest[2 * NI + ND:2 * NI + 2 * ND]
    acc_sh = rest[2 * NI + 2 * ND]
    c = lax.axis_index("c")
    t = lax.axis_index("s")
    r0 = t * ROWS_PER_TILE
    wbase = t * NW

    def load_idx(i, w):
        pltpu.async_copy(rc_hbm.at[wbase + w], idx[i], isem[i])

    def wait_idx(i):
        pltpu.make_async_copy(rc_hbm.at[0], idx[i], isem[i]).wait()

    def run(y_ref):
        pltpu.sync_copy(y_ref.at[pl.ds(r0, ROWS_PER_TILE)],
                        acc_sh.at[pl.ds(r0, ROWS_PER_TILE)])
        plsc.subcore_barrier()

        def gather(b, i):
            pltpu.async_copy(y_ref.at[idx[i].at[0]], bufs[b], gsem[b])

        def wait_gather(b):
            pltpu.make_async_copy(y_ref.at[pl.ds(0, K)], bufs[b],
                                  gsem[b]).wait()

        def win(i, w, gather_next, load_next):
            b = i % ND
            wait_gather(b)
            pltpu.sync_copy(bufs[b], acc_sh.at[idx[i].at[1]], add=True)
            if gather_next:
                j = (i + 2) % NI
                wait_idx(j)
                gather(b, j)
            if load_next:
                load_idx(i, w + NI)

        for i in range(NI):
            load_idx(i, i)
        for b in range(ND):
            wait_idx(b)
            gather(b, b)

        def body(q, carry):
            for i in range(NI):
                win(i, q * NI + i, True, True)
            return carry

        lax.fori_loop(0, NW // NI - 1, body, 0)
        for i in range(NI):
            win(i, NW - NI + i, i < 2, False)

        plsc.subcore_barrier()
        pltpu.sync_copy(acc_sh.at[pl.ds(r0, ROWS_PER_TILE)],
                        out_hbm.at[c, pl.ds(r0, ROWS_PER_TILE)])

    @pl.when(c == 0)
    def _():
        run(y0_hbm)

    @pl.when(c == 1)
    def _():
        run(y1_hbm)


def _mm_first_body(x_ref, w_ref, deg_ref, y0_ref, y1_ref):
    d = deg_ref[0] + deg_ref[1]
    dinv = lax.rsqrt(d)
    z = jnp.dot(x_ref[...], w_ref[...], preferred_element_type=jnp.float32)
    y = z * dinv
    y0_ref[...] = y[:, :HALF]
    y1_ref[...] = y[:, HALF:]


def _mm_mid_body(a0_ref, a1_ref, b_ref, wa_ref, wb_ref, deg_ref,
                 y0_ref, y1_ref):
    d = deg_ref[0] + deg_ref[1]
    dinv = lax.rsqrt(d)
    b = b_ref[...]
    h0 = jnp.maximum(a0_ref[...] * dinv + b[:, :HALF], 0.0)
    h1 = jnp.maximum(a1_ref[...] * dinv + b[:, HALF:], 0.0)
    z = (jnp.dot(h0, wa_ref[...], preferred_element_type=jnp.float32)
         + jnp.dot(h1, wb_ref[...], preferred_element_type=jnp.float32))
    y = z * dinv
    y0_ref[...] = y[:, :HALF]
    y1_ref[...] = y[:, HALF:]


def _pool_body(a0_ref, a1_ref, b_ref, deg_ref, batch_ref, out_ref,
               sums, cnts):
    i = pl.program_id(0)

    @pl.when(i == 0)
    def _():
        sums[...] = jnp.zeros_like(sums)
        cnts[...] = jnp.zeros_like(cnts)

    d = deg_ref[0] + deg_ref[1]
    dinv = lax.rsqrt(d)
    b = b_ref[...]
    h0 = jnp.maximum(a0_ref[...] * dinv + b[:, :HALF], 0.0)
    h1 = jnp.maximum(a1_ref[...] * dinv + b[:, HALF:], 0.0)
    h = jnp.concatenate([h0, h1], axis=1)
    gids = lax.broadcasted_iota(jnp.int32, (1, G), 1)
    mask = (batch_ref[...] == gids).astype(jnp.float32)
    dn = (((0,), (0,)), ((), ()))
    sums[...] += lax.dot_general(mask, h, dn,
                                 preferred_element_type=jnp.float32)
    cnts[...] += lax.dot_general(mask, jnp.ones((MB, HALF), jnp.float32), dn,
                                 preferred_element_type=jnp.float32)

    @pl.when(i == MGRID - 1)
    def _():
        out_ref[...] = sums[...] / jnp.maximum(cnts[:, :1], 1.0)


def _mm_first(x, w, deg):
    return pl.pallas_call(
        _mm_first_body,
        grid=(MGRID,),
        in_specs=[
            pl.BlockSpec((MB, HALF), lambda i: (i, 0)),
            pl.BlockSpec((HALF, HID), lambda i: (0, 0)),
            pl.BlockSpec((NC, MB, 1), lambda i: (0, i, 0)),
        ],
        out_specs=[
            pl.BlockSpec((MB, HALF), lambda i: (i, 0)),
            pl.BlockSpec((MB, HALF), lambda i: (i, 0)),
        ],
        out_shape=[
            jax.ShapeDtypeStruct((NP, HALF), jnp.float32),
            jax.ShapeDtypeStruct((NP, HALF), jnp.float32),
        ],
    )(x, w, deg)


def _mm_mid(a0, a1, b, wa, wb, deg):
    return pl.pallas_call(
        _mm_mid_body,
        grid=(MGRID,),
        in_specs=[
            pl.BlockSpec((MB, HALF), lambda i: (i, 0)),
            pl.BlockSpec((MB, HALF), lambda i: (i, 0)),
            pl.BlockSpec((1, HID), lambda i: (0, 0)),
            pl.BlockSpec((HALF, HID), lambda i: (0, 0)),
            pl.BlockSpec((HALF, HID), lambda i: (0, 0)),
            pl.BlockSpec((NC, MB, 1), lambda i: (0, i, 0)),
        ],
        out_specs=[
            pl.BlockSpec((MB, HALF), lambda i: (i, 0)),
            pl.BlockSpec((MB, HALF), lambda i: (i, 0)),
        ],
        out_shape=[
            jax.ShapeDtypeStruct((NP, HALF), jnp.float32),
            jax.ShapeDtypeStruct((NP, HALF), jnp.float32),
        ],
    )(a0, a1, b, wa, wb, deg)


def _pool(a0, a1, b, deg, batch2d):
    return pl.pallas_call(
        _pool_body,
        grid=(MGRID,),
        in_specs=[
            pl.BlockSpec((MB, HALF), lambda i: (i, 0)),
            pl.BlockSpec((MB, HALF), lambda i: (i, 0)),
            pl.BlockSpec((1, HID), lambda i: (0, 0)),
            pl.BlockSpec((NC, MB, 1), lambda i: (0, i, 0)),
            pl.BlockSpec((MB, 1), lambda i: (i, 0)),
        ],
        out_specs=pl.BlockSpec((G, HID), lambda i: (0, 0)),
        out_shape=jax.ShapeDtypeStruct((G, HID), jnp.float32),
        scratch_shapes=[
            pltpu.VMEM((G, HID), jnp.float32),
            pltpu.VMEM((G, HALF), jnp.float32),
        ],
    )(a0, a1, b, deg, batch2d)


def kernel(x, edge_index, batch, W0, b0, W1, b1, W2, b2, W3, b3, W4, b4):
    pad = (jnp.arange(EP - E, dtype=jnp.int32) % (NP - N)) + N
    row = jnp.concatenate([edge_index[0], pad]).reshape(NS * NW, 1, K)
    col = jnp.concatenate([edge_index[1], pad]).reshape(NS * NW, 1, K)
    rc = jnp.concatenate([row, col], axis=1)
    init2 = jnp.stack([jnp.ones((NP,), jnp.float32),
                       jnp.zeros((NP,), jnp.float32)])

    deg2 = _deg_sc(rc, init2)
    deg = deg2.reshape(NC, NP, 1)

    xp = jnp.pad(x, ((0, NP - N), (0, 0)))
    batch2d = jnp.pad(batch, (0, NP - N), constant_values=G).reshape(NP, 1)
    Ws = [W1, W2, W3, W4]
    bs = [b0.reshape(1, HID), b1.reshape(1, HID), b2.reshape(1, HID),
          b3.reshape(1, HID), b4.reshape(1, HID)]

    y0, y1 = _mm_first(xp, W0, deg)
    a = _agg_sc(y0, y1, rc)
    for i in range(4):
        y0, y1 = _mm_mid(a[0], a[1], bs[i], Ws[i][:HALF], Ws[i][HALF:], deg)
        a = _agg_sc(y0, y1, rc)
    return _pool(a[0], a[1], bs[4], deg, batch2d)


# speedup vs baseline: 19.6654x; 1.0577x over previous
import functools

import jax
import jax.numpy as jnp
from jax import lax
from jax.experimental import pallas as pl
from jax.experimental.pallas import tpu as pltpu
from jax.experimental.pallas import tpu_sc as plsc

N = 10000
E = 320000
G = 64
HID = 256
HALF = 128
NC = 2
NS = 16
NP = 10240
ROWS_PER_TILE = NP // NS
DEG_PER_TILE = NP // NS
EP = 327680
K = 128
MB = 1024
MGRID = NP // MB

_sc_mesh = plsc.VectorSubcoreMesh(core_axis_name="c", subcore_axis_name="s")


@functools.partial(
    pl.kernel,
    out_type=jax.ShapeDtypeStruct((NC, NP), jnp.float32),
    mesh=_sc_mesh,
    scratch_types=(
        [pltpu.VMEM((K,), jnp.int32)] * 4
        + [pltpu.SemaphoreType.DMA] * 4
        + [pltpu.VMEM((K,), jnp.float32),
           pltpu.VMEM_SHARED((NP,), jnp.float32)]
    ),
)
def _deg_sc(col_hbm, init_hbm, deg_hbm, *rest):
    idxs = rest[:4]
    isem = rest[4:8]
    ones_v = rest[8]
    deg_sh = rest[9]
    c = lax.axis_index("c")
    t = lax.axis_index("s")
    r0 = t * DEG_PER_TILE
    pltpu.sync_copy(init_hbm.at[c, pl.ds(r0, DEG_PER_TILE)],
                    deg_sh.at[pl.ds(r0, DEG_PER_TILE)])
    pltpu.sync_copy(init_hbm.at[0, pl.ds(0, K)], ones_v)
    plsc.subcore_barrier()

    wpt = NW // NC
    base = (c * NS + t) * wpt

    def load_idx(i, w):
        pltpu.async_copy(col_hbm.at[base + w, 1], idxs[i], isem[i])

    def wait_idx(i):
        pltpu.make_async_copy(col_hbm.at[0, 1], idxs[i], isem[i]).wait()

    def win(i, w, load_next):
        wait_idx(i)
        pltpu.sync_copy(ones_v, deg_sh.at[idxs[i]], add=True)
        if load_next:
            load_idx(i, w + 4)

    for i in range(4):
        load_idx(i, i)

    def body(q, carry):
        for i in range(4):
            win(i, q * 4 + i, True)
        return carry

    lax.fori_loop(0, wpt // 4 - 1, body, 0)
    for i in range(4):
        win(i, wpt - 4 + i, False)
    plsc.subcore_barrier()
    pltpu.sync_copy(deg_sh.at[pl.ds(r0, DEG_PER_TILE)],
                    deg_hbm.at[c, pl.ds(r0, DEG_PER_TILE)])


NW = EP // (NS * K)
NI = 4
ND = 2

@functools.partial(
    pl.kernel,
    out_type=(jax.ShapeDtypeStruct((NP, HALF), jnp.float32),
              jax.ShapeDtypeStruct((NP, HALF), jnp.float32)),
    mesh=_sc_mesh,
    scratch_types=(
        [pltpu.VMEM((2, K), jnp.int32)] * NI
        + [pltpu.VMEM((K, HALF), jnp.float32)] * ND
        + [pltpu.SemaphoreType.DMA] * (NI + ND)
        + [pltpu.VMEM_SHARED((NP, HALF), jnp.float32)]
    ),
)
def _agg_sc(y0_hbm, y1_hbm, rc_hbm, out0_hbm, out1_hbm, *rest):
    idx = rest[:NI]
    bufs = rest[NI:NI + ND]
    isem = rest[NI + ND:2 * NI + ND]
    gsem = rest[2 * NI + ND:2 * NI + 2 * ND]
    acc_sh = rest[2 * NI + 2 * ND]
    c = lax.axis_index("c")
    t = lax.axis_index("s")
    r0 = t * ROWS_PER_TILE
    wbase = t * NW

    def load_idx(i, w):
        pltpu.async_copy(rc_hbm.at[wbase + w], idx[i], isem[i])

    def wait_idx(i):
        pltpu.make_async_copy(rc_hbm.at[0], idx[i], isem[i]).wait()

    def run(y_ref):
        pltpu.sync_copy(y_ref.at[pl.ds(r0, ROWS_PER_TILE)],
                        acc_sh.at[pl.ds(r0, ROWS_PER_TILE)])
        plsc.subcore_barrier()

        def gather(b, i):
            pltpu.async_copy(y_ref.at[idx[i].at[0]], bufs[b], gsem[b])

        def wait_gather(b):
            pltpu.make_async_copy(y_ref.at[pl.ds(0, K)], bufs[b],
                                  gsem[b]).wait()

        def win(i, w, gather_next, load_next):
            b = i % ND
            wait_gather(b)
            pltpu.sync_copy(bufs[b], acc_sh.at[idx[i].at[1]], add=True)
            if gather_next:
                j = (i + 2) % NI
                wait_idx(j)
                gather(b, j)
            if load_next:
                load_idx(i, w + NI)

        for i in range(NI):
            load_idx(i, i)
        for b in range(ND):
            wait_idx(b)
            gather(b, b)

        def body(q, carry):
            for i in range(NI):
                win(i, q * NI + i, True, True)
            return carry

        lax.fori_loop(0, NW // NI - 1, body, 0)
        for i in range(NI):
            win(i, NW - NI + i, i < 2, False)

        plsc.subcore_barrier()

    @pl.when(c == 0)
    def _():
        run(y0_hbm)
        pltpu.sync_copy(acc_sh.at[pl.ds(r0, ROWS_PER_TILE)],
                        out0_hbm.at[pl.ds(r0, ROWS_PER_TILE)])

    @pl.when(c == 1)
    def _():
        run(y1_hbm)
        pltpu.sync_copy(acc_sh.at[pl.ds(r0, ROWS_PER_TILE)],
                        out1_hbm.at[pl.ds(r0, ROWS_PER_TILE)])


def _mm_first_body(x_ref, w_ref, deg_ref, y0_ref, y1_ref):
    d = deg_ref[0] + deg_ref[1]
    dinv = lax.rsqrt(d)
    z = jnp.dot(x_ref[...], w_ref[...], preferred_element_type=jnp.float32)
    y = z * dinv
    y0_ref[...] = y[:, :HALF]
    y1_ref[...] = y[:, HALF:]


def _mm_mid_body(a0_ref, a1_ref, b_ref, wa_ref, wb_ref, deg_ref,
                 y0_ref, y1_ref):
    d = deg_ref[0] + deg_ref[1]
    dinv = lax.rsqrt(d)
    b = b_ref[...]
    h0 = jnp.maximum(a0_ref[...] * dinv + b[:, :HALF], 0.0)
    h1 = jnp.maximum(a1_ref[...] * dinv + b[:, HALF:], 0.0)
    z = (jnp.dot(h0, wa_ref[...], preferred_element_type=jnp.float32)
         + jnp.dot(h1, wb_ref[...], preferred_element_type=jnp.float32))
    y = z * dinv
    y0_ref[...] = y[:, :HALF]
    y1_ref[...] = y[:, HALF:]


def _pool_body(a0_ref, a1_ref, b_ref, deg_ref, batch_ref, out_ref,
               sums, cnts):
    i = pl.program_id(0)

    @pl.when(i == 0)
    def _():
        sums[...] = jnp.zeros_like(sums)
        cnts[...] = jnp.zeros_like(cnts)

    d = deg_ref[0] + deg_ref[1]
    dinv = lax.rsqrt(d)
    b = b_ref[...]
    h0 = jnp.maximum(a0_ref[...] * dinv + b[:, :HALF], 0.0)
    h1 = jnp.maximum(a1_ref[...] * dinv + b[:, HALF:], 0.0)
    h = jnp.concatenate([h0, h1], axis=1)
    gids = lax.broadcasted_iota(jnp.int32, (1, G), 1)
    mask = (batch_ref[...] == gids).astype(jnp.float32)
    dn = (((0,), (0,)), ((), ()))
    sums[...] += lax.dot_general(mask, h, dn,
                                 preferred_element_type=jnp.float32)
    cnts[...] += lax.dot_general(mask, jnp.ones((MB, HALF), jnp.float32), dn,
                                 preferred_element_type=jnp.float32)

    @pl.when(i == MGRID - 1)
    def _():
        out_ref[...] = sums[...] / jnp.maximum(cnts[:, :1], 1.0)


def _mm_first(x, w, deg):
    return pl.pallas_call(
        _mm_first_body,
        grid=(MGRID,),
        in_specs=[
            pl.BlockSpec((MB, HALF), lambda i: (i, 0)),
            pl.BlockSpec((HALF, HID), lambda i: (0, 0)),
            pl.BlockSpec((NC, MB, 1), lambda i: (0, i, 0)),
        ],
        out_specs=[
            pl.BlockSpec((MB, HALF), lambda i: (i, 0)),
            pl.BlockSpec((MB, HALF), lambda i: (i, 0)),
        ],
        out_shape=[
            jax.ShapeDtypeStruct((NP, HALF), jnp.float32),
            jax.ShapeDtypeStruct((NP, HALF), jnp.float32),
        ],
    )(x, w, deg)


def _mm_mid(a0, a1, b, wa, wb, deg):
    return pl.pallas_call(
        _mm_mid_body,
        grid=(MGRID,),
        in_specs=[
            pl.BlockSpec((MB, HALF), lambda i: (i, 0)),
            pl.BlockSpec((MB, HALF), lambda i: (i, 0)),
            pl.BlockSpec((1, HID), lambda i: (0, 0)),
            pl.BlockSpec((HALF, HID), lambda i: (0, 0)),
            pl.BlockSpec((HALF, HID), lambda i: (0, 0)),
            pl.BlockSpec((NC, MB, 1), lambda i: (0, i, 0)),
        ],
        out_specs=[
            pl.BlockSpec((MB, HALF), lambda i: (i, 0)),
            pl.BlockSpec((MB, HALF), lambda i: (i, 0)),
        ],
        out_shape=[
            jax.ShapeDtypeStruct((NP, HALF), jnp.float32),
            jax.ShapeDtypeStruct((NP, HALF), jnp.float32),
        ],
    )(a0, a1, b, wa, wb, deg)


def _pool(a0, a1, b, deg, batch2d):
    return pl.pallas_call(
        _pool_body,
        grid=(MGRID,),
        in_specs=[
            pl.BlockSpec((MB, HALF), lambda i: (i, 0)),
            pl.BlockSpec((MB, HALF), lambda i: (i, 0)),
            pl.BlockSpec((1, HID), lambda i: (0, 0)),
            pl.BlockSpec((NC, MB, 1), lambda i: (0, i, 0)),
            pl.BlockSpec((MB, 1), lambda i: (i, 0)),
        ],
        out_specs=pl.BlockSpec((G, HID), lambda i: (0, 0)),
        out_shape=jax.ShapeDtypeStruct((G, HID), jnp.float32),
        scratch_shapes=[
            pltpu.VMEM((G, HID), jnp.float32),
            pltpu.VMEM((G, HALF), jnp.float32),
        ],
    )(a0, a1, b, deg, batch2d)


def kernel(x, edge_index, batch, W0, b0, W1, b1, W2, b2, W3, b3, W4, b4):
    pad = (jnp.arange(EP - E, dtype=jnp.int32) % (NP - N)) + N
    row = jnp.concatenate([edge_index[0], pad]).reshape(NS * NW, 1, K)
    col = jnp.concatenate([edge_index[1], pad]).reshape(NS * NW, 1, K)
    rc = jnp.concatenate([row, col], axis=1)
    init2 = jnp.stack([jnp.ones((NP,), jnp.float32),
                       jnp.zeros((NP,), jnp.float32)])

    deg2 = _deg_sc(rc, init2)
    deg = deg2.reshape(NC, NP, 1)

    xp = jnp.pad(x, ((0, NP - N), (0, 0)))
    batch2d = jnp.pad(batch, (0, NP - N), constant_values=G).reshape(NP, 1)
    Ws = [W1, W2, W3, W4]
    bs = [b0.reshape(1, HID), b1.reshape(1, HID), b2.reshape(1, HID),
          b3.reshape(1, HID), b4.reshape(1, HID)]

    y0, y1 = _mm_first(xp, W0, deg)
    a0, a1 = _agg_sc(y0, y1, rc)
    for i in range(4):
        y0, y1 = _mm_mid(a0, a1, bs[i], Ws[i][:HALF], Ws[i][HALF:], deg)
        a0, a1 = _agg_sc(y0, y1, rc)
    return _pool(a0, a1, bs[4], deg, batch2d)


# speedup vs baseline: 19.9418x; 1.0141x over previous
import functools

import jax
import jax.numpy as jnp
from jax import lax
from jax.experimental import pallas as pl
from jax.experimental.pallas import tpu as pltpu
from jax.experimental.pallas import tpu_sc as plsc

N = 10000
E = 320000
G = 64
HID = 256
HALF = 128
NC = 2
NS = 16
NP = 10240
ROWS_PER_TILE = NP // NS
DEG_PER_TILE = NP // NS
EP = 327680
K = 128
MB = 2048
MGRID = NP // MB

_sc_mesh = plsc.VectorSubcoreMesh(core_axis_name="c", subcore_axis_name="s")


@functools.partial(
    pl.kernel,
    out_type=jax.ShapeDtypeStruct((NC, NP), jnp.float32),
    mesh=_sc_mesh,
    scratch_types=(
        [pltpu.VMEM((K,), jnp.int32)] * 4
        + [pltpu.SemaphoreType.DMA] * 4
        + [pltpu.VMEM((K,), jnp.float32),
           pltpu.VMEM_SHARED((NP,), jnp.float32)]
    ),
)
def _deg_sc(col_hbm, init_hbm, deg_hbm, *rest):
    idxs = rest[:4]
    isem = rest[4:8]
    ones_v = rest[8]
    deg_sh = rest[9]
    c = lax.axis_index("c")
    t = lax.axis_index("s")
    r0 = t * DEG_PER_TILE
    pltpu.sync_copy(init_hbm.at[c, pl.ds(r0, DEG_PER_TILE)],
                    deg_sh.at[pl.ds(r0, DEG_PER_TILE)])
    pltpu.sync_copy(init_hbm.at[0, pl.ds(0, K)], ones_v)
    plsc.subcore_barrier()

    wpt = NW // NC
    base = (c * NS + t) * wpt

    def load_idx(i, w):
        pltpu.async_copy(col_hbm.at[base + w, 1], idxs[i], isem[i])

    def wait_idx(i):
        pltpu.make_async_copy(col_hbm.at[0, 1], idxs[i], isem[i]).wait()

    def win(i, w, load_next):
        wait_idx(i)
        pltpu.sync_copy(ones_v, deg_sh.at[idxs[i]], add=True)
        if load_next:
            load_idx(i, w + 4)

    for i in range(4):
        load_idx(i, i)

    def body(q, carry):
        for i in range(4):
            win(i, q * 4 + i, True)
        return carry

    lax.fori_loop(0, wpt // 4 - 1, body, 0)
    for i in range(4):
        win(i, wpt - 4 + i, False)
    plsc.subcore_barrier()
    pltpu.sync_copy(deg_sh.at[pl.ds(r0, DEG_PER_TILE)],
                    deg_hbm.at[c, pl.ds(r0, DEG_PER_TILE)])


NW = EP // (NS * K)
NI = 4
ND = 2

@functools.partial(
    pl.kernel,
    out_type=(jax.ShapeDtypeStruct((NP, HALF), jnp.float32),
              jax.ShapeDtypeStruct((NP, HALF), jnp.float32)),
    mesh=_sc_mesh,
    scratch_types=(
        [pltpu.VMEM((2, K), jnp.int32)] * NI
        + [pltpu.VMEM((K, HALF), jnp.float32)] * ND
        + [pltpu.SemaphoreType.DMA] * (NI + ND)
        + [pltpu.VMEM_SHARED((NP, HALF), jnp.float32)]
    ),
)
def _agg_sc(y0_hbm, y1_hbm, rc_hbm, out0_hbm, out1_hbm, *rest):
    idx = rest[:NI]
    bufs = rest[NI:NI + ND]
    isem = rest[NI + ND:2 * NI + ND]
    gsem = rest[2 * NI + ND:2 * NI + 2 * ND]
    acc_sh = rest[2 * NI + 2 * ND]
    c = lax.axis_index("c")
    t = lax.axis_index("s")
    r0 = t * ROWS_PER_TILE
    wbase = t * NW

    def load_idx(i, w):
        pltpu.async_copy(rc_hbm.at[wbase + w], idx[i], isem[i])

    def wait_idx(i):
        pltpu.make_async_copy(rc_hbm.at[0], idx[i], isem[i]).wait()

    def run(y_ref):
        pltpu.sync_copy(y_ref.at[pl.ds(r0, ROWS_PER_TILE)],
                        acc_sh.at[pl.ds(r0, ROWS_PER_TILE)])
        plsc.subcore_barrier()

        def gather(b, i):
            pltpu.async_copy(y_ref.at[idx[i].at[0]], bufs[b], gsem[b])

        def wait_gather(b):
            pltpu.make_async_copy(y_ref.at[pl.ds(0, K)], bufs[b],
                                  gsem[b]).wait()

        def win(i, w, gather_next, load_next):
            b = i % ND
            wait_gather(b)
            pltpu.sync_copy(bufs[b], acc_sh.at[idx[i].at[1]], add=True)
            if gather_next:
                j = (i + 2) % NI
                wait_idx(j)
                gather(b, j)
            if load_next:
                load_idx(i, w + NI)

        for i in range(NI):
            load_idx(i, i)
        for b in range(ND):
            wait_idx(b)
            gather(b, b)

        def body(q, carry):
            for i in range(NI):
                win(i, q * NI + i, True, True)
            return carry

        lax.fori_loop(0, NW // NI - 1, body, 0)
        for i in range(NI):
            win(i, NW - NI + i, i < 2, False)

        plsc.subcore_barrier()

    @pl.when(c == 0)
    def _():
        run(y0_hbm)
        pltpu.sync_copy(acc_sh.at[pl.ds(r0, ROWS_PER_TILE)],
                        out0_hbm.at[pl.ds(r0, ROWS_PER_TILE)])

    @pl.when(c == 1)
    def _():
        run(y1_hbm)
        pltpu.sync_copy(acc_sh.at[pl.ds(r0, ROWS_PER_TILE)],
                        out1_hbm.at[pl.ds(r0, ROWS_PER_TILE)])


def _mm_first_body(x_ref, w_ref, deg_ref, y0_ref, y1_ref):
    d = deg_ref[0] + deg_ref[1]
    dinv = lax.rsqrt(d)
    z = jnp.dot(x_ref[...], w_ref[...], preferred_element_type=jnp.float32)
    y = z * dinv
    y0_ref[...] = y[:, :HALF]
    y1_ref[...] = y[:, HALF:]


def _mm_mid_body(a0_ref, a1_ref, b_ref, wa_ref, wb_ref, deg_ref,
                 y0_ref, y1_ref):
    d = deg_ref[0] + deg_ref[1]
    dinv = lax.rsqrt(d)
    b = b_ref[...]
    h0 = jnp.maximum(a0_ref[...] * dinv + b[:, :HALF], 0.0)
    h1 = jnp.maximum(a1_ref[...] * dinv + b[:, HALF:], 0.0)
    z = (jnp.dot(h0, wa_ref[...], preferred_element_type=jnp.float32)
         + jnp.dot(h1, wb_ref[...], preferred_element_type=jnp.float32))
    y = z * dinv
    y0_ref[...] = y[:, :HALF]
    y1_ref[...] = y[:, HALF:]


def _pool_body(a0_ref, a1_ref, b_ref, deg_ref, batch_ref, out_ref,
               sums, cnts):
    i = pl.program_id(0)

    @pl.when(i == 0)
    def _():
        sums[...] = jnp.zeros_like(sums)
        cnts[...] = jnp.zeros_like(cnts)

    d = deg_ref[0] + deg_ref[1]
    dinv = lax.rsqrt(d)
    b = b_ref[...]
    h0 = jnp.maximum(a0_ref[...] * dinv + b[:, :HALF], 0.0)
    h1 = jnp.maximum(a1_ref[...] * dinv + b[:, HALF:], 0.0)
    h = jnp.concatenate([h0, h1], axis=1)
    gids = lax.broadcasted_iota(jnp.int32, (1, G), 1)
    mask = (batch_ref[...] == gids).astype(jnp.float32)
    dn = (((0,), (0,)), ((), ()))
    sums[...] += lax.dot_general(mask, h, dn,
                                 preferred_element_type=jnp.float32)
    cnts[...] += lax.dot_general(mask, jnp.ones((MB, HALF), jnp.float32), dn,
                                 preferred_element_type=jnp.float32)

    @pl.when(i == MGRID - 1)
    def _():
        out_ref[...] = sums[...] / jnp.maximum(cnts[:, :1], 1.0)


def _mm_first(x, w, deg):
    return pl.pallas_call(
        _mm_first_body,
        grid=(MGRID,),
        in_specs=[
            pl.BlockSpec((MB, HALF), lambda i: (i, 0)),
            pl.BlockSpec((HALF, HID), lambda i: (0, 0)),
            pl.BlockSpec((NC, MB, 1), lambda i: (0, i, 0)),
        ],
        out_specs=[
            pl.BlockSpec((MB, HALF), lambda i: (i, 0)),
            pl.BlockSpec((MB, HALF), lambda i: (i, 0)),
        ],
        out_shape=[
            jax.ShapeDtypeStruct((NP, HALF), jnp.float32),
            jax.ShapeDtypeStruct((NP, HALF), jnp.float32),
        ],
    )(x, w, deg)


def _mm_mid(a0, a1, b, wa, wb, deg):
    return pl.pallas_call(
        _mm_mid_body,
        grid=(MGRID,),
        in_specs=[
            pl.BlockSpec((MB, HALF), lambda i: (i, 0)),
            pl.BlockSpec((MB, HALF), lambda i: (i, 0)),
            pl.BlockSpec((1, HID), lambda i: (0, 0)),
            pl.BlockSpec((HALF, HID), lambda i: (0, 0)),
            pl.BlockSpec((HALF, HID), lambda i: (0, 0)),
            pl.BlockSpec((NC, MB, 1), lambda i: (0, i, 0)),
        ],
        out_specs=[
            pl.BlockSpec((MB, HALF), lambda i: (i, 0)),
            pl.BlockSpec((MB, HALF), lambda i: (i, 0)),
        ],
        out_shape=[
            jax.ShapeDtypeStruct((NP, HALF), jnp.float32),
            jax.ShapeDtypeStruct((NP, HALF), jnp.float32),
        ],
    )(a0, a1, b, wa, wb, deg)


def _pool(a0, a1, b, deg, batch2d):
    return pl.pallas_call(
        _pool_body,
        grid=(MGRID,),
        in_specs=[
            pl.BlockSpec((MB, HALF), lambda i: (i, 0)),
            pl.BlockSpec((MB, HALF), lambda i: (i, 0)),
            pl.BlockSpec((1, HID), lambda i: (0, 0)),
            pl.BlockSpec((NC, MB, 1), lambda i: (0, i, 0)),
            pl.BlockSpec((MB, 1), lambda i: (i, 0)),
        ],
        out_specs=pl.BlockSpec((G, HID), lambda i: (0, 0)),
        out_shape=jax.ShapeDtypeStruct((G, HID), jnp.float32),
        scratch_shapes=[
            pltpu.VMEM((G, HID), jnp.float32),
            pltpu.VMEM((G, HALF), jnp.float32),
        ],
    )(a0, a1, b, deg, batch2d)


def kernel(x, edge_index, batch, W0, b0, W1, b1, W2, b2, W3, b3, W4, b4):
    pad = (jnp.arange(EP - E, dtype=jnp.int32) % (NP - N)) + N
    row = jnp.concatenate([edge_index[0], pad]).reshape(NS * NW, 1, K)
    col = jnp.concatenate([edge_index[1], pad]).reshape(NS * NW, 1, K)
    rc = jnp.concatenate([row, col], axis=1)
    init2 = jnp.stack([jnp.ones((NP,), jnp.float32),
                       jnp.zeros((NP,), jnp.float32)])

    deg2 = _deg_sc(rc, init2)
    deg = deg2.reshape(NC, NP, 1)

    xp = jnp.pad(x, ((0, NP - N), (0, 0)))
    batch2d = jnp.pad(batch, (0, NP - N), constant_values=G).reshape(NP, 1)
    Ws = [W1, W2, W3, W4]
    bs = [b0.reshape(1, HID), b1.reshape(1, HID), b2.reshape(1, HID),
          b3.reshape(1, HID), b4.reshape(1, HID)]

    y0, y1 = _mm_first(xp, W0, deg)
    a0, a1 = _agg_sc(y0, y1, rc)
    for i in range(4):
        y0, y1 = _mm_mid(a0, a1, bs[i], Ws[i][:HALF], Ws[i][HALF:], deg)
        a0, a1 = _agg_sc(y0, y1, rc)
    return _pool(a0, a1, bs[4], deg, batch2d)


# speedup vs baseline: 19.9982x; 1.0028x over previous
import functools

import jax
import jax.numpy as jnp
from jax import lax
from jax.experimental import pallas as pl
from jax.experimental.pallas import tpu as pltpu
from jax.experimental.pallas import tpu_sc as plsc

N = 10000
E = 320000
G = 64
HID = 256
HALF = 128
NC = 2
NS = 16
NP = 10240
ROWS_PER_TILE = NP // NS
DEG_PER_TILE = NP // NS
EP = 327680
K = 128
MB = 2560
MGRID = NP // MB

_sc_mesh = plsc.VectorSubcoreMesh(core_axis_name="c", subcore_axis_name="s")


@functools.partial(
    pl.kernel,
    out_type=jax.ShapeDtypeStruct((NC, NP), jnp.float32),
    mesh=_sc_mesh,
    scratch_types=(
        [pltpu.VMEM((K,), jnp.int32)] * 4
        + [pltpu.SemaphoreType.DMA] * 4
        + [pltpu.VMEM((K,), jnp.float32),
           pltpu.VMEM_SHARED((NP,), jnp.float32)]
    ),
)
def _deg_sc(col_hbm, init_hbm, deg_hbm, *rest):
    idxs = rest[:4]
    isem = rest[4:8]
    ones_v = rest[8]
    deg_sh = rest[9]
    c = lax.axis_index("c")
    t = lax.axis_index("s")
    r0 = t * DEG_PER_TILE
    pltpu.sync_copy(init_hbm.at[c, pl.ds(r0, DEG_PER_TILE)],
                    deg_sh.at[pl.ds(r0, DEG_PER_TILE)])
    pltpu.sync_copy(init_hbm.at[0, pl.ds(0, K)], ones_v)
    plsc.subcore_barrier()

    wpt = NW // NC
    base = (c * NS + t) * wpt

    def load_idx(i, w):
        pltpu.async_copy(col_hbm.at[base + w, 1], idxs[i], isem[i])

    def wait_idx(i):
        pltpu.make_async_copy(col_hbm.at[0, 1], idxs[i], isem[i]).wait()

    def win(i, w, load_next):
        wait_idx(i)
        pltpu.sync_copy(ones_v, deg_sh.at[idxs[i]], add=True)
        if load_next:
            load_idx(i, w + 4)

    for i in range(4):
        load_idx(i, i)

    def body(q, carry):
        for i in range(4):
            win(i, q * 4 + i, True)
        return carry

    lax.fori_loop(0, wpt // 4 - 1, body, 0)
    for i in range(4):
        win(i, wpt - 4 + i, False)
    plsc.subcore_barrier()
    pltpu.sync_copy(deg_sh.at[pl.ds(r0, DEG_PER_TILE)],
                    deg_hbm.at[c, pl.ds(r0, DEG_PER_TILE)])


NW = EP // (NS * K)
NI = 4
ND = 2

@functools.partial(
    pl.kernel,
    out_type=(jax.ShapeDtypeStruct((NP, HALF), jnp.float32),
              jax.ShapeDtypeStruct((NP, HALF), jnp.float32)),
    mesh=_sc_mesh,
    scratch_types=(
        [pltpu.VMEM((2, K), jnp.int32)] * NI
        + [pltpu.VMEM((K, HALF), jnp.float32)] * ND
        + [pltpu.SemaphoreType.DMA] * (NI + ND)
        + [pltpu.VMEM_SHARED((NP, HALF), jnp.float32)]
    ),
)
def _agg_sc(y0_hbm, y1_hbm, rc_hbm, out0_hbm, out1_hbm, *rest):
    idx = rest[:NI]
    bufs = rest[NI:NI + ND]
    isem = rest[NI + ND:2 * NI + ND]
    gsem = rest[2 * NI + ND:2 * NI + 2 * ND]
    acc_sh = rest[2 * NI + 2 * ND]
    c = lax.axis_index("c")
    t = lax.axis_index("s")
    r0 = t * ROWS_PER_TILE
    wbase = t * NW

    def load_idx(i, w):
        pltpu.async_copy(rc_hbm.at[wbase + w], idx[i], isem[i])

    def wait_idx(i):
        pltpu.make_async_copy(rc_hbm.at[0], idx[i], isem[i]).wait()

    def run(y_ref):
        pltpu.sync_copy(y_ref.at[pl.ds(r0, ROWS_PER_TILE)],
                        acc_sh.at[pl.ds(r0, ROWS_PER_TILE)])
        plsc.subcore_barrier()

        def gather(b, i):
            pltpu.async_copy(y_ref.at[idx[i].at[0]], bufs[b], gsem[b])

        def wait_gather(b):
            pltpu.make_async_copy(y_ref.at[pl.ds(0, K)], bufs[b],
                                  gsem[b]).wait()

        def win(i, w, gather_next, load_next):
            b = i % ND
            wait_gather(b)
            pltpu.sync_copy(bufs[b], acc_sh.at[idx[i].at[1]], add=True)
            if gather_next:
                j = (i + 2) % NI
                wait_idx(j)
                gather(b, j)
            if load_next:
                load_idx(i, w + NI)

        for i in range(NI):
            load_idx(i, i)
        for b in range(ND):
            wait_idx(b)
            gather(b, b)

        def body(q, carry):
            for i in range(NI):
                win(i, q * NI + i, True, True)
            return carry

        lax.fori_loop(0, NW // NI - 1, body, 0)
        for i in range(NI):
            win(i, NW - NI + i, i < 2, False)

        plsc.subcore_barrier()

    @pl.when(c == 0)
    def _():
        run(y0_hbm)
        pltpu.sync_copy(acc_sh.at[pl.ds(r0, ROWS_PER_TILE)],
                        out0_hbm.at[pl.ds(r0, ROWS_PER_TILE)])

    @pl.when(c == 1)
    def _():
        run(y1_hbm)
        pltpu.sync_copy(acc_sh.at[pl.ds(r0, ROWS_PER_TILE)],
                        out1_hbm.at[pl.ds(r0, ROWS_PER_TILE)])


def _mm_first_body(x_ref, w_ref, deg_ref, y0_ref, y1_ref):
    d = deg_ref[0] + deg_ref[1]
    dinv = lax.rsqrt(d)
    z = jnp.dot(x_ref[...], w_ref[...], preferred_element_type=jnp.float32)
    y = z * dinv
    y0_ref[...] = y[:, :HALF]
    y1_ref[...] = y[:, HALF:]


def _mm_mid_body(a0_ref, a1_ref, b_ref, wa_ref, wb_ref, deg_ref,
                 y0_ref, y1_ref):
    d = deg_ref[0] + deg_ref[1]
    dinv = lax.rsqrt(d)
    b = b_ref[...]
    h0 = jnp.maximum(a0_ref[...] * dinv + b[:, :HALF], 0.0)
    h1 = jnp.maximum(a1_ref[...] * dinv + b[:, HALF:], 0.0)
    z = (jnp.dot(h0, wa_ref[...], preferred_element_type=jnp.float32)
         + jnp.dot(h1, wb_ref[...], preferred_element_type=jnp.float32))
    y = z * dinv
    y0_ref[...] = y[:, :HALF]
    y1_ref[...] = y[:, HALF:]


def _pool_body(a0_ref, a1_ref, b_ref, deg_ref, batch_ref, out_ref,
               sums, cnts):
    i = pl.program_id(0)

    @pl.when(i == 0)
    def _():
        sums[...] = jnp.zeros_like(sums)
        cnts[...] = jnp.zeros_like(cnts)

    d = deg_ref[0] + deg_ref[1]
    dinv = lax.rsqrt(d)
    b = b_ref[...]
    h0 = jnp.maximum(a0_ref[...] * dinv + b[:, :HALF], 0.0)
    h1 = jnp.maximum(a1_ref[...] * dinv + b[:, HALF:], 0.0)
    h = jnp.concatenate([h0, h1], axis=1)
    gids = lax.broadcasted_iota(jnp.int32, (1, G), 1)
    mask = (batch_ref[...] == gids).astype(jnp.float32)
    dn = (((0,), (0,)), ((), ()))
    sums[...] += lax.dot_general(mask, h, dn,
                                 preferred_element_type=jnp.float32)
    cnts[...] += lax.dot_general(mask, jnp.ones((MB, HALF), jnp.float32), dn,
                                 preferred_element_type=jnp.float32)

    @pl.when(i == MGRID - 1)
    def _():
        out_ref[...] = sums[...] / jnp.maximum(cnts[:, :1], 1.0)


def _mm_first(x, w, deg):
    return pl.pallas_call(
        _mm_first_body,
        grid=(MGRID,),
        in_specs=[
            pl.BlockSpec((MB, HALF), lambda i: (i, 0)),
            pl.BlockSpec((HALF, HID), lambda i: (0, 0)),
            pl.BlockSpec((NC, MB, 1), lambda i: (0, i, 0)),
        ],
        out_specs=[
            pl.BlockSpec((MB, HALF), lambda i: (i, 0)),
            pl.BlockSpec((MB, HALF), lambda i: (i, 0)),
        ],
        out_shape=[
            jax.ShapeDtypeStruct((NP, HALF), jnp.float32),
            jax.ShapeDtypeStruct((NP, HALF), jnp.float32),
        ],
    )(x, w, deg)


def _mm_mid(a0, a1, b, wa, wb, deg):
    return pl.pallas_call(
        _mm_mid_body,
        grid=(MGRID,),
        in_specs=[
            pl.BlockSpec((MB, HALF), lambda i: (i, 0)),
            pl.BlockSpec((MB, HALF), lambda i: (i, 0)),
            pl.BlockSpec((1, HID), lambda i: (0, 0)),
            pl.BlockSpec((HALF, HID), lambda i: (0, 0)),
            pl.BlockSpec((HALF, HID), lambda i: (0, 0)),
            pl.BlockSpec((NC, MB, 1), lambda i: (0, i, 0)),
        ],
        out_specs=[
            pl.BlockSpec((MB, HALF), lambda i: (i, 0)),
            pl.BlockSpec((MB, HALF), lambda i: (i, 0)),
        ],
        out_shape=[
            jax.ShapeDtypeStruct((NP, HALF), jnp.float32),
            jax.ShapeDtypeStruct((NP, HALF), jnp.float32),
        ],
    )(a0, a1, b, wa, wb, deg)


def _pool(a0, a1, b, deg, batch2d):
    return pl.pallas_call(
        _pool_body,
        grid=(MGRID,),
        in_specs=[
            pl.BlockSpec((MB, HALF), lambda i: (i, 0)),
            pl.BlockSpec((MB, HALF), lambda i: (i, 0)),
            pl.BlockSpec((1, HID), lambda i: (0, 0)),
            pl.BlockSpec((NC, MB, 1), lambda i: (0, i, 0)),
            pl.BlockSpec((MB, 1), lambda i: (i, 0)),
        ],
        out_specs=pl.BlockSpec((G, HID), lambda i: (0, 0)),
        out_shape=jax.ShapeDtypeStruct((G, HID), jnp.float32),
        scratch_shapes=[
            pltpu.VMEM((G, HID), jnp.float32),
            pltpu.VMEM((G, HALF), jnp.float32),
        ],
    )(a0, a1, b, deg, batch2d)


def kernel(x, edge_index, batch, W0, b0, W1, b1, W2, b2, W3, b3, W4, b4):
    pad = (jnp.arange(EP - E, dtype=jnp.int32) % (NP - N)) + N
    row = jnp.concatenate([edge_index[0], pad]).reshape(NS * NW, 1, K)
    col = jnp.concatenate([edge_index[1], pad]).reshape(NS * NW, 1, K)
    rc = jnp.concatenate([row, col], axis=1)
    init2 = jnp.stack([jnp.ones((NP,), jnp.float32),
                       jnp.zeros((NP,), jnp.float32)])

    deg2 = _deg_sc(rc, init2)
    deg = deg2.reshape(NC, NP, 1)

    xp = jnp.pad(x, ((0, NP - N), (0, 0)))
    batch2d = jnp.pad(batch, (0, NP - N), constant_values=G).reshape(NP, 1)
    Ws = [W1, W2, W3, W4]
    bs = [b0.reshape(1, HID), b1.reshape(1, HID), b2.reshape(1, HID),
          b3.reshape(1, HID), b4.reshape(1, HID)]

    y0, y1 = _mm_first(xp, W0, deg)
    a0, a1 = _agg_sc(y0, y1, rc)
    for i in range(4):
        y0, y1 = _mm_mid(a0, a1, bs[i], Ws[i][:HALF], Ws[i][HALF:], deg)
        a0, a1 = _agg_sc(y0, y1, rc)
    return _pool(a0, a1, bs[4], deg, batch2d)


# speedup vs baseline: 20.0934x; 1.0048x over previous
import functools

import jax
import jax.numpy as jnp
from jax import lax
from jax.experimental import pallas as pl
from jax.experimental.pallas import tpu as pltpu
from jax.experimental.pallas import tpu_sc as plsc

N = 10000
E = 320000
G = 64
HID = 256
HALF = 128
NC = 2
NS = 16
NP = 10240
ROWS_PER_TILE = NP // NS
DEG_PER_TILE = NP // NS
EP = 327680
K = 128
MB = 2560
MGRID = NP // MB

_sc_mesh = plsc.VectorSubcoreMesh(core_axis_name="c", subcore_axis_name="s")


@functools.partial(
    pl.kernel,
    out_type=jax.ShapeDtypeStruct((NC, NP), jnp.float32),
    mesh=_sc_mesh,
    scratch_types=(
        [pltpu.VMEM((K,), jnp.int32)] * 4
        + [pltpu.SemaphoreType.DMA] * 4
        + [pltpu.VMEM((K,), jnp.float32),
           pltpu.VMEM_SHARED((NP,), jnp.float32)]
    ),
)
def _deg_sc(col_hbm, init_hbm, deg_hbm, *rest):
    idxs = rest[:4]
    isem = rest[4:8]
    ones_v = rest[8]
    deg_sh = rest[9]
    c = lax.axis_index("c")
    t = lax.axis_index("s")
    r0 = t * DEG_PER_TILE
    pltpu.sync_copy(init_hbm.at[c, pl.ds(r0, DEG_PER_TILE)],
                    deg_sh.at[pl.ds(r0, DEG_PER_TILE)])
    pltpu.sync_copy(init_hbm.at[0, pl.ds(0, K)], ones_v)
    plsc.subcore_barrier()

    wpt = NW // NC
    base = (c * NS + t) * wpt

    def load_idx(i, w):
        pltpu.async_copy(col_hbm.at[base + w, 1], idxs[i], isem[i])

    def wait_idx(i):
        pltpu.make_async_copy(col_hbm.at[0, 1], idxs[i], isem[i]).wait()

    def win(i, w, load_next):
        wait_idx(i)
        pltpu.sync_copy(ones_v, deg_sh.at[idxs[i]], add=True)
        if load_next:
            load_idx(i, w + 4)

    for i in range(4):
        load_idx(i, i)

    def body(q, carry):
        for i in range(4):
            win(i, q * 4 + i, True)
        return carry

    lax.fori_loop(0, wpt // 4 - 1, body, 0)
    for i in range(4):
        win(i, wpt - 4 + i, False)
    plsc.subcore_barrier()
    pltpu.sync_copy(deg_sh.at[pl.ds(r0, DEG_PER_TILE)],
                    deg_hbm.at[c, pl.ds(r0, DEG_PER_TILE)])


NW = EP // (NS * K)
NI = 4
ND = 2

@functools.partial(
    pl.kernel,
    out_type=(jax.ShapeDtypeStruct((NP, HALF), jnp.float32),
              jax.ShapeDtypeStruct((NP, HALF), jnp.float32)),
    mesh=_sc_mesh,
    scratch_types=(
        [pltpu.VMEM((2, K), jnp.int32)] * NI
        + [pltpu.VMEM((K, HALF), jnp.float32)] * ND
        + [pltpu.SemaphoreType.DMA] * (NI + ND)
        + [pltpu.VMEM_SHARED((NP, HALF), jnp.float32)]
    ),
)
def _agg_sc(y0_hbm, y1_hbm, rc_hbm, out0_hbm, out1_hbm, *rest):
    idx = rest[:NI]
    bufs = rest[NI:NI + ND]
    isem = rest[NI + ND:2 * NI + ND]
    gsem = rest[2 * NI + ND:2 * NI + 2 * ND]
    acc_sh = rest[2 * NI + 2 * ND]
    c = lax.axis_index("c")
    t = lax.axis_index("s")
    r0 = t * ROWS_PER_TILE
    wbase = t * NW

    def load_idx(i, w):
        pltpu.async_copy(rc_hbm.at[wbase + w], idx[i], isem[i])

    def wait_idx(i):
        pltpu.make_async_copy(rc_hbm.at[0], idx[i], isem[i]).wait()

    def run(y_ref):
        def gather(b, i):
            pltpu.async_copy(y_ref.at[idx[i].at[0]], bufs[b], gsem[b])

        def wait_gather(b):
            pltpu.make_async_copy(y_ref.at[pl.ds(0, K)], bufs[b],
                                  gsem[b]).wait()

        def win(i, w, gather_next, load_next):
            b = i % ND
            wait_gather(b)
            pltpu.sync_copy(bufs[b], acc_sh.at[idx[i].at[1]], add=True)
            if gather_next:
                j = (i + 2) % NI
                wait_idx(j)
                gather(b, j)
            if load_next:
                load_idx(i, w + NI)

        for i in range(NI):
            load_idx(i, i)
        for b in range(ND):
            wait_idx(b)
            gather(b, b)
        pltpu.sync_copy(y_ref.at[pl.ds(r0, ROWS_PER_TILE)],
                        acc_sh.at[pl.ds(r0, ROWS_PER_TILE)])
        plsc.subcore_barrier()

        def body(q, carry):
            for i in range(NI):
                win(i, q * NI + i, True, True)
            return carry

        lax.fori_loop(0, NW // NI - 1, body, 0)
        for i in range(NI):
            win(i, NW - NI + i, i < 2, False)

        plsc.subcore_barrier()

    @pl.when(c == 0)
    def _():
        run(y0_hbm)
        pltpu.sync_copy(acc_sh.at[pl.ds(r0, ROWS_PER_TILE)],
                        out0_hbm.at[pl.ds(r0, ROWS_PER_TILE)])

    @pl.when(c == 1)
    def _():
        run(y1_hbm)
        pltpu.sync_copy(acc_sh.at[pl.ds(r0, ROWS_PER_TILE)],
                        out1_hbm.at[pl.ds(r0, ROWS_PER_TILE)])


def _mm_first_body(x_ref, w_ref, z_ref):
    z_ref[...] = jnp.dot(x_ref[...], w_ref[...],
                         preferred_element_type=jnp.float32)


def _scale_split_body(z_ref, deg_ref, y0_ref, y1_ref):
    d = deg_ref[0] + deg_ref[1]
    dinv = lax.rsqrt(d)
    y = z_ref[...] * dinv
    y0_ref[...] = y[:, :HALF]
    y1_ref[...] = y[:, HALF:]


def _mm_mid_body(a0_ref, a1_ref, b_ref, wa_ref, wb_ref, deg_ref,
                 y0_ref, y1_ref):
    d = deg_ref[0] + deg_ref[1]
    dinv = lax.rsqrt(d)
    b = b_ref[...]
    h0 = jnp.maximum(a0_ref[...] * dinv + b[:, :HALF], 0.0)
    h1 = jnp.maximum(a1_ref[...] * dinv + b[:, HALF:], 0.0)
    z = (jnp.dot(h0, wa_ref[...], preferred_element_type=jnp.float32)
         + jnp.dot(h1, wb_ref[...], preferred_element_type=jnp.float32))
    y = z * dinv
    y0_ref[...] = y[:, :HALF]
    y1_ref[...] = y[:, HALF:]


def _pool_body(a0_ref, a1_ref, b_ref, deg_ref, batch_ref, out_ref,
               sums, cnts):
    i = pl.program_id(0)

    @pl.when(i == 0)
    def _():
        sums[...] = jnp.zeros_like(sums)
        cnts[...] = jnp.zeros_like(cnts)

    d = deg_ref[0] + deg_ref[1]
    dinv = lax.rsqrt(d)
    b = b_ref[...]
    h0 = jnp.maximum(a0_ref[...] * dinv + b[:, :HALF], 0.0)
    h1 = jnp.maximum(a1_ref[...] * dinv + b[:, HALF:], 0.0)
    h = jnp.concatenate([h0, h1], axis=1)
    gids = lax.broadcasted_iota(jnp.int32, (1, G), 1)
    mask = (batch_ref[...] == gids).astype(jnp.float32)
    dn = (((0,), (0,)), ((), ()))
    sums[...] += lax.dot_general(mask, h, dn,
                                 preferred_element_type=jnp.float32)
    cnts[...] += lax.dot_general(mask, jnp.ones((MB, HALF), jnp.float32), dn,
                                 preferred_element_type=jnp.float32)

    @pl.when(i == MGRID - 1)
    def _():
        out_ref[...] = sums[...] / jnp.maximum(cnts[:, :1], 1.0)


def _mm_first(x, w):
    return pl.pallas_call(
        _mm_first_body,
        grid=(MGRID,),
        in_specs=[
            pl.BlockSpec((MB, HALF), lambda i: (i, 0)),
            pl.BlockSpec((HALF, HID), lambda i: (0, 0)),
        ],
        out_specs=pl.BlockSpec((MB, HID), lambda i: (i, 0)),
        out_shape=jax.ShapeDtypeStruct((NP, HID), jnp.float32),
    )(x, w)


def _scale_split(z, deg):
    return pl.pallas_call(
        _scale_split_body,
        grid=(MGRID,),
        in_specs=[
            pl.BlockSpec((MB, HID), lambda i: (i, 0)),
            pl.BlockSpec((NC, MB, 1), lambda i: (0, i, 0)),
        ],
        out_specs=[
            pl.BlockSpec((MB, HALF), lambda i: (i, 0)),
            pl.BlockSpec((MB, HALF), lambda i: (i, 0)),
        ],
        out_shape=[
            jax.ShapeDtypeStruct((NP, HALF), jnp.float32),
            jax.ShapeDtypeStruct((NP, HALF), jnp.float32),
        ],
    )(z, deg)


def _mm_mid(a0, a1, b, wa, wb, deg):
    return pl.pallas_call(
        _mm_mid_body,
        grid=(MGRID,),
        in_specs=[
            pl.BlockSpec((MB, HALF), lambda i: (i, 0)),
            pl.BlockSpec((MB, HALF), lambda i: (i, 0)),
            pl.BlockSpec((1, HID), lambda i: (0, 0)),
            pl.BlockSpec((HALF, HID), lambda i: (0, 0)),
            pl.BlockSpec((HALF, HID), lambda i: (0, 0)),
            pl.BlockSpec((NC, MB, 1), lambda i: (0, i, 0)),
        ],
        out_specs=[
            pl.BlockSpec((MB, HALF), lambda i: (i, 0)),
            pl.BlockSpec((MB, HALF), lambda i: (i, 0)),
        ],
        out_shape=[
            jax.ShapeDtypeStruct((NP, HALF), jnp.float32),
            jax.ShapeDtypeStruct((NP, HALF), jnp.float32),
        ],
    )(a0, a1, b, wa, wb, deg)


def _pool(a0, a1, b, deg, batch2d):
    return pl.pallas_call(
        _pool_body,
        grid=(MGRID,),
        in_specs=[
            pl.BlockSpec((MB, HALF), lambda i: (i, 0)),
            pl.BlockSpec((MB, HALF), lambda i: (i, 0)),
            pl.BlockSpec((1, HID), lambda i: (0, 0)),
            pl.BlockSpec((NC, MB, 1), lambda i: (0, i, 0)),
            pl.BlockSpec((MB, 1), lambda i: (i, 0)),
        ],
        out_specs=pl.BlockSpec((G, HID), lambda i: (0, 0)),
        out_shape=jax.ShapeDtypeStruct((G, HID), jnp.float32),
        scratch_shapes=[
            pltpu.VMEM((G, HID), jnp.float32),
            pltpu.VMEM((G, HALF), jnp.float32),
        ],
    )(a0, a1, b, deg, batch2d)


def kernel(x, edge_index, batch, W0, b0, W1, b1, W2, b2, W3, b3, W4, b4):
    pad = (jnp.arange(EP - E, dtype=jnp.int32) % (NP - N)) + N
    row = jnp.concatenate([edge_index[0], pad]).reshape(NS * NW, 1, K)
    col = jnp.concatenate([edge_index[1], pad]).reshape(NS * NW, 1, K)
    rc = jnp.concatenate([row, col], axis=1)
    init2 = jnp.stack([jnp.ones((NP,), jnp.float32),
                       jnp.zeros((NP,), jnp.float32)])

    deg2 = _deg_sc(rc, init2)
    deg = deg2.reshape(NC, NP, 1)

    xp = jnp.pad(x, ((0, NP - N), (0, 0)))
    batch2d = jnp.pad(batch, (0, NP - N), constant_values=G).reshape(NP, 1)
    Ws = [W1, W2, W3, W4]
    bs = [b0.reshape(1, HID), b1.reshape(1, HID), b2.reshape(1, HID),
          b3.reshape(1, HID), b4.reshape(1, HID)]

    z = _mm_first(xp, W0)
    y0, y1 = _scale_split(z, deg)
    a0, a1 = _agg_sc(y0, y1, rc)
    for i in range(4):
        y0, y1 = _mm_mid(a0, a1, bs[i], Ws[i][:HALF], Ws[i][HALF:], deg)
        a0, a1 = _agg_sc(y0, y1, rc)
    return _pool(a0, a1, bs[4], deg, batch2d)


# speedup vs baseline: 20.3954x; 1.0150x over previous
import functools

import jax
import jax.numpy as jnp
from jax import lax
from jax.experimental import pallas as pl
from jax.experimental.pallas import tpu as pltpu
from jax.experimental.pallas import tpu_sc as plsc

N = 10000
E = 320000
G = 64
HID = 256
HALF = 128
NC = 2
NS = 16
NP = 10240
ROWS_PER_TILE = NP // NS
DEG_PER_TILE = NP // NS
EP = 323584
K = 128
MB = 5120
MGRID = NP // MB

_sc_mesh = plsc.VectorSubcoreMesh(core_axis_name="c", subcore_axis_name="s")


@functools.partial(
    pl.kernel,
    out_type=jax.ShapeDtypeStruct((NC, NP), jnp.float32),
    mesh=_sc_mesh,
    scratch_types=(
        [pltpu.VMEM((K,), jnp.int32)] * 4
        + [pltpu.SemaphoreType.DMA] * 4
        + [pltpu.VMEM((K,), jnp.float32),
           pltpu.VMEM_SHARED((NP,), jnp.float32)]
    ),
)
def _deg_sc(col_hbm, init_hbm, deg_hbm, *rest):
    idxs = rest[:4]
    isem = rest[4:8]
    ones_v = rest[8]
    deg_sh = rest[9]
    c = lax.axis_index("c")
    t = lax.axis_index("s")
    r0 = t * DEG_PER_TILE
    pltpu.sync_copy(init_hbm.at[c, pl.ds(r0, DEG_PER_TILE)],
                    deg_sh.at[pl.ds(r0, DEG_PER_TILE)])
    pltpu.sync_copy(init_hbm.at[0, pl.ds(0, K)], ones_v)
    plsc.subcore_barrier()

    wpt = NW // NC
    base = (c * NS + t) * wpt

    def load_idx(i, w):
        pltpu.async_copy(col_hbm.at[base + w, 1], idxs[i], isem[i])

    def wait_idx(i):
        pltpu.make_async_copy(col_hbm.at[0, 1], idxs[i], isem[i]).wait()

    def win(i, w, load_next):
        wait_idx(i)
        pltpu.sync_copy(ones_v, deg_sh.at[idxs[i]], add=True)
        if load_next:
            load_idx(i, w + 4)

    for i in range(4):
        load_idx(i, i)

    def body(q, carry):
        for i in range(4):
            win(i, q * 4 + i, True)
        return carry

    ntail = wpt % 4 + 4
    lax.fori_loop(0, (wpt - ntail) // 4, body, 0)
    for i in range(ntail):
        win(i % 4, wpt - ntail + i, i < ntail - 4)
    plsc.subcore_barrier()
    pltpu.sync_copy(deg_sh.at[pl.ds(r0, DEG_PER_TILE)],
                    deg_hbm.at[c, pl.ds(r0, DEG_PER_TILE)])


NW = EP // (NS * K)
NI = 4
ND = 2

@functools.partial(
    pl.kernel,
    out_type=(jax.ShapeDtypeStruct((NP, HALF), jnp.float32),
              jax.ShapeDtypeStruct((NP, HALF), jnp.float32)),
    mesh=_sc_mesh,
    scratch_types=(
        [pltpu.VMEM((2, K), jnp.int32)] * NI
        + [pltpu.VMEM((K, HALF), jnp.float32)] * ND
        + [pltpu.SemaphoreType.DMA] * (NI + ND)
        + [pltpu.VMEM_SHARED((NP, HALF), jnp.float32)]
    ),
)
def _agg_sc(y0_hbm, y1_hbm, rc_hbm, out0_hbm, out1_hbm, *rest):
    idx = rest[:NI]
    bufs = rest[NI:NI + ND]
    isem = rest[NI + ND:2 * NI + ND]
    gsem = rest[2 * NI + ND:2 * NI + 2 * ND]
    acc_sh = rest[2 * NI + 2 * ND]
    c = lax.axis_index("c")
    t = lax.axis_index("s")
    r0 = t * ROWS_PER_TILE
    wbase = t * NW

    def load_idx(i, w):
        pltpu.async_copy(rc_hbm.at[wbase + w], idx[i], isem[i])

    def wait_idx(i):
        pltpu.make_async_copy(rc_hbm.at[0], idx[i], isem[i]).wait()

    def run(y_ref):
        def gather(b, i):
            pltpu.async_copy(y_ref.at[idx[i].at[0]], bufs[b], gsem[b])

        def wait_gather(b):
            pltpu.make_async_copy(y_ref.at[pl.ds(0, K)], bufs[b],
                                  gsem[b]).wait()

        def win(i, w, gather_next, load_next):
            b = i % ND
            wait_gather(b)
            pltpu.sync_copy(bufs[b], acc_sh.at[idx[i].at[1]], add=True)
            if gather_next:
                j = (i + 2) % NI
                wait_idx(j)
                gather(b, j)
            if load_next:
                load_idx(i, w + NI)

        for i in range(NI):
            load_idx(i, i)
        for b in range(ND):
            wait_idx(b)
            gather(b, b)
        pltpu.sync_copy(y_ref.at[pl.ds(r0, ROWS_PER_TILE)],
                        acc_sh.at[pl.ds(r0, ROWS_PER_TILE)])
        plsc.subcore_barrier()

        def body(q, carry):
            for i in range(NI):
                win(i, q * NI + i, True, True)
            return carry

        ntail = NW % NI + NI
        lax.fori_loop(0, (NW - ntail) // NI, body, 0)
        for i in range(ntail):
            win(i % NI, NW - ntail + i, i < ntail - 2, i < ntail - 4)

        plsc.subcore_barrier()

    @pl.when(c == 0)
    def _():
        run(y0_hbm)
        pltpu.sync_copy(acc_sh.at[pl.ds(r0, ROWS_PER_TILE)],
                        out0_hbm.at[pl.ds(r0, ROWS_PER_TILE)])

    @pl.when(c == 1)
    def _():
        run(y1_hbm)
        pltpu.sync_copy(acc_sh.at[pl.ds(r0, ROWS_PER_TILE)],
                        out1_hbm.at[pl.ds(r0, ROWS_PER_TILE)])


def _mm_first_body(x_ref, w_ref, deg_ref, y0_ref, y1_ref):
    d = deg_ref[0] + deg_ref[1]
    dinv = lax.rsqrt(d)
    z = jnp.dot(x_ref[...], w_ref[...], preferred_element_type=jnp.float32)
    y = z * dinv
    y0_ref[...] = y[:, :HALF]
    y1_ref[...] = y[:, HALF:]


def _mm_mid_body(a0_ref, a1_ref, b_ref, wa_ref, wb_ref, deg_ref,
                 y0_ref, y1_ref):
    d = deg_ref[0] + deg_ref[1]
    dinv = lax.rsqrt(d)
    b = b_ref[...]
    h0 = jnp.maximum(a0_ref[...] * dinv + b[:, :HALF], 0.0)
    h1 = jnp.maximum(a1_ref[...] * dinv + b[:, HALF:], 0.0)
    z = (jnp.dot(h0, wa_ref[...], preferred_element_type=jnp.float32)
         + jnp.dot(h1, wb_ref[...], preferred_element_type=jnp.float32))
    y = z * dinv
    y0_ref[...] = y[:, :HALF]
    y1_ref[...] = y[:, HALF:]


def _pool_body(a0_ref, a1_ref, b_ref, deg_ref, batch_ref, out_ref,
               sums, cnts):
    i = pl.program_id(0)

    @pl.when(i == 0)
    def _():
        sums[...] = jnp.zeros_like(sums)
        cnts[...] = jnp.zeros_like(cnts)

    d = deg_ref[0] + deg_ref[1]
    dinv = lax.rsqrt(d)
    b = b_ref[...]
    h0 = jnp.maximum(a0_ref[...] * dinv + b[:, :HALF], 0.0)
    h1 = jnp.maximum(a1_ref[...] * dinv + b[:, HALF:], 0.0)
    h = jnp.concatenate([h0, h1], axis=1)
    gids = lax.broadcasted_iota(jnp.int32, (1, G), 1)
    mask = (batch_ref[...] == gids).astype(jnp.float32)
    dn = (((0,), (0,)), ((), ()))
    sums[...] += lax.dot_general(mask, h, dn,
                                 preferred_element_type=jnp.float32)
    cnts[...] += lax.dot_general(mask, jnp.ones((MB, HALF), jnp.float32), dn,
                                 preferred_element_type=jnp.float32)

    @pl.when(i == MGRID - 1)
    def _():
        out_ref[...] = sums[...] / jnp.maximum(cnts[:, :1], 1.0)


def _mm_first(x, w, deg):
    return pl.pallas_call(
        _mm_first_body,
        grid=(MGRID,),
        in_specs=[
            pl.BlockSpec((MB, HALF), lambda i: (i, 0)),
            pl.BlockSpec((HALF, HID), lambda i: (0, 0)),
            pl.BlockSpec((NC, MB, 1), lambda i: (0, i, 0)),
        ],
        out_specs=[
            pl.BlockSpec((MB, HALF), lambda i: (i, 0)),
            pl.BlockSpec((MB, HALF), lambda i: (i, 0)),
        ],
        out_shape=[
            jax.ShapeDtypeStruct((NP, HALF), jnp.float32),
            jax.ShapeDtypeStruct((NP, HALF), jnp.float32),
        ],
    )(x, w, deg)


def _mm_mid(a0, a1, b, wa, wb, deg):
    return pl.pallas_call(
        _mm_mid_body,
        grid=(MGRID,),
        in_specs=[
            pl.BlockSpec((MB, HALF), lambda i: (i, 0)),
            pl.BlockSpec((MB, HALF), lambda i: (i, 0)),
            pl.BlockSpec((1, HID), lambda i: (0, 0)),
            pl.BlockSpec((HALF, HID), lambda i: (0, 0)),
            pl.BlockSpec((HALF, HID), lambda i: (0, 0)),
            pl.BlockSpec((NC, MB, 1), lambda i: (0, i, 0)),
        ],
        out_specs=[
            pl.BlockSpec((MB, HALF), lambda i: (i, 0)),
            pl.BlockSpec((MB, HALF), lambda i: (i, 0)),
        ],
        out_shape=[
            jax.ShapeDtypeStruct((NP, HALF), jnp.float32),
            jax.ShapeDtypeStruct((NP, HALF), jnp.float32),
        ],
    )(a0, a1, b, wa, wb, deg)


def _pool(a0, a1, b, deg, batch2d):
    return pl.pallas_call(
        _pool_body,
        grid=(MGRID,),
        in_specs=[
            pl.BlockSpec((MB, HALF), lambda i: (i, 0)),
            pl.BlockSpec((MB, HALF), lambda i: (i, 0)),
            pl.BlockSpec((1, HID), lambda i: (0, 0)),
            pl.BlockSpec((NC, MB, 1), lambda i: (0, i, 0)),
            pl.BlockSpec((MB, 1), lambda i: (i, 0)),
        ],
        out_specs=pl.BlockSpec((G, HID), lambda i: (0, 0)),
        out_shape=jax.ShapeDtypeStruct((G, HID), jnp.float32),
        scratch_shapes=[
            pltpu.VMEM((G, HID), jnp.float32),
            pltpu.VMEM((G, HALF), jnp.float32),
        ],
    )(a0, a1, b, deg, batch2d)


def kernel(x, edge_index, batch, W0, b0, W1, b1, W2, b2, W3, b3, W4, b4):
    pad = (jnp.arange(EP - E, dtype=jnp.int32) % (NP - N)) + N
    row = jnp.concatenate([edge_index[0], pad]).reshape(NS * NW, 1, K)
    col = jnp.concatenate([edge_index[1], pad]).reshape(NS * NW, 1, K)
    rc = jnp.concatenate([row, col], axis=1)
    init2 = jnp.stack([jnp.ones((NP,), jnp.float32),
                       jnp.zeros((NP,), jnp.float32)])

    deg2 = _deg_sc(rc, init2)
    deg = deg2.reshape(NC, NP, 1)

    xp = jnp.pad(x, ((0, NP - N), (0, 0)))
    batch2d = jnp.pad(batch, (0, NP - N), constant_values=G).reshape(NP, 1)
    Ws = [W1, W2, W3, W4]
    bs = [b0.reshape(1, HID), b1.reshape(1, HID), b2.reshape(1, HID),
          b3.reshape(1, HID), b4.reshape(1, HID)]

    y0, y1 = _mm_first(xp, W0, deg)
    a0, a1 = _agg_sc(y0, y1, rc)
    for i in range(4):
        y0, y1 = _mm_mid(a0, a1, bs[i], Ws[i][:HALF], Ws[i][HALF:], deg)
        a0, a1 = _agg_sc(y0, y1, rc)
    return _pool(a0, a1, bs[4], deg, batch2d)
